# baseline (speedup 1.0000x reference)
import functools

import jax
import jax.numpy as jnp
from jax import lax
from jax.experimental import pallas as pl
from jax.experimental.pallas import tpu as pltpu
from jax.experimental.pallas import tpu_sc as plsc

NC, NS = 2, 16
NW = NC * NS
BATCH, SEQ, EMBED = 1024, 200, 300
VOCAB = 1000
VPAD = 1024
ECH = 25
NECH = EMBED // ECH
BT = BATCH // 128
NBLK = (SEQ // 8) * BT
NUNIT = NBLK * NECH
L = 16


def _sc_kernel(xt, wt_flat):
    mesh = plsc.VectorSubcoreMesh(core_axis_name="c", subcore_axis_name="s")

    @functools.partial(
        pl.kernel,
        out_type=jax.ShapeDtypeStruct((EMBED, SEQ, BATCH), jnp.float32),
        mesh=mesh,
        scratch_types=[
            pltpu.VMEM((8, 128), jnp.int32),
            pltpu.VMEM((8, 128), jnp.int32),
            pltpu.VMEM((ECH * VPAD,), jnp.float32),
            pltpu.VMEM((ECH, 8, 128), jnp.float32),
            pltpu.VMEM((ECH, 8, 128), jnp.float32),
            pltpu.SemaphoreType.DMA,
            pltpu.SemaphoreType.DMA,
            pltpu.SemaphoreType.DMA,
            pltpu.SemaphoreType.DMA,
        ],
        compiler_params=pltpu.CompilerParams(
            use_tc_tiling_on_sc=True, needs_layout_passes=False
        ),
    )
    def k(xt_hbm, wt_hbm, out_hbm, ib0, ib1, wt, buf0, buf1,
          isem0, isem1, osem0, osem1):
        wid = lax.axis_index("s") * NC + lax.axis_index("c")
        u0 = wid * NUNIT // NW
        u1 = (wid + 1) * NUNIT // NW
        cnt = u1 - u0
        ibs = (ib0, ib1)
        isem = (isem0, isem1)
        bufs = (buf0, buf1)
        osem = (osem0, osem1)

        def idx_start(u, p):
            blk = u % NBLK
            rg = blk // BT
            bt = blk % BT
            pltpu.async_copy(
                xt_hbm.at[pl.ds(rg * 8, 8), pl.ds(bt * 128, 128)],
                ibs[p], isem[p],
            )

        def idx_wait(p):
            pltpu.make_async_copy(
                xt_hbm.at[pl.ds(0, 8), pl.ds(0, 128)], ibs[p], isem[p]
            ).wait()

        def out_wait(p):
            pltpu.make_async_copy(
                bufs[p],
                out_hbm.at[pl.ds(0, ECH), pl.ds(0, 8), pl.ds(0, 128)],
                osem[p],
            ).wait()

        idx_start(u0, 0)

        def step(t, p, cprev):
            u = u0 + t
            live = u < u1
            c = u // NBLK
            blk = u % NBLK
            rg = blk // BT
            bt = blk % BT

            @pl.when(live & (c != cprev))
            def _():
                pltpu.sync_copy(
                    wt_hbm.at[pl.ds(c * (ECH * VPAD), ECH * VPAD)], wt
                )

            @pl.when(live)
            def _():
                idx_wait(p)

                @pl.when(u + 1 < u1)
                def _():
                    idx_start(u + 1, p ^ 1)

                @pl.when(t >= 2)
                def _():
                    out_wait(p)

                @plsc.parallel_loop(0, 64, unroll=2)
                def col(sj):
                    s = sj // 8
                    j = sj % 8
                    iv = ibs[p][s, pl.ds(j * L, L)]
                    vals = [
                        plsc.load_gather(wt.at[pl.ds(e * VPAD, VPAD)], [iv])
                        for e in range(ECH)
                    ]
                    for e in range(ECH):
                        bufs[p][e, s, pl.ds(j * L, L)] = vals[e]

                pltpu.async_copy(
                    bufs[p],
                    out_hbm.at[
                        pl.ds(c * ECH, ECH),
                        pl.ds(rg * 8, 8),
                        pl.ds(bt * 128, 128),
                    ],
                    osem[p],
                )

            return jnp.where(live, c, cprev)

        def pair(i, cprev):
            cprev = step(i * 2, 0, cprev)
            cprev = step(i * 2 + 1, 1, cprev)
            return cprev

        lax.fori_loop(0, (cnt + 1) // 2, pair, jnp.int32(-1))
        out_wait(0)
        out_wait(1)

    return k(xt, wt_flat)


def kernel(x, W):
    xt = x.T
    wt = jnp.pad(W.T, ((0, 0), (0, VPAD - VOCAB)))
    wt_flat = wt.reshape(-1)
    r = _sc_kernel(xt, wt_flat)
    return jnp.transpose(r, (2, 1, 0))


# speedup vs baseline: 1.2907x; 1.2907x over previous
import functools

import jax
import jax.numpy as jnp
from jax import lax
from jax.experimental import pallas as pl
from jax.experimental.pallas import tpu as pltpu
from jax.experimental.pallas import tpu_sc as plsc

NC, NS = 2, 16
NW = NC * NS
BATCH, SEQ, EMBED = 1024, 200, 300
VOCAB = 1000
VPAD = 1024
ECH = 30
NECH = EMBED // ECH
BT = BATCH // 128
NBLK = (SEQ // 8) * BT
NUNIT = NBLK * NECH
L = 16


def _sc_kernel(xt, wt_flat):
    mesh = plsc.VectorSubcoreMesh(core_axis_name="c", subcore_axis_name="s")

    @functools.partial(
        pl.kernel,
        out_type=jax.ShapeDtypeStruct((EMBED, SEQ, BATCH), jnp.float32),
        mesh=mesh,
        scratch_types=[
            pltpu.VMEM((8, 128), jnp.int32),
            pltpu.VMEM((8, 128), jnp.int32),
            pltpu.VMEM((ECH * VPAD,), jnp.float32),
            pltpu.VMEM((ECH, 8, 128), jnp.float32),
            pltpu.VMEM((ECH, 8, 128), jnp.float32),
            pltpu.SemaphoreType.DMA,
            pltpu.SemaphoreType.DMA,
            pltpu.SemaphoreType.DMA,
            pltpu.SemaphoreType.DMA,
        ],
        compiler_params=pltpu.CompilerParams(
            use_tc_tiling_on_sc=True, needs_layout_passes=False
        ),
    )
    def k(xt_hbm, wt_hbm, out_hbm, ib0, ib1, wt, buf0, buf1,
          isem0, isem1, osem0, osem1):
        wid = lax.axis_index("s") * NC + lax.axis_index("c")
        u0 = wid * NUNIT // NW
        u1 = (wid + 1) * NUNIT // NW
        cnt = u1 - u0
        ibs = (ib0, ib1)
        isem = (isem0, isem1)
        bufs = (buf0, buf1)
        osem = (osem0, osem1)

        def idx_start(u, p):
            blk = u % NBLK
            rg = blk // BT
            bt = blk % BT
            pltpu.async_copy(
                xt_hbm.at[pl.ds(rg * 8, 8), pl.ds(bt * 128, 128)],
                ibs[p], isem[p],
            )

        def idx_wait(p):
            pltpu.make_async_copy(
                xt_hbm.at[pl.ds(0, 8), pl.ds(0, 128)], ibs[p], isem[p]
            ).wait()

        def out_wait(p):
            pltpu.make_async_copy(
                bufs[p],
                out_hbm.at[pl.ds(0, ECH), pl.ds(0, 8), pl.ds(0, 128)],
                osem[p],
            ).wait()

        idx_start(u0, 0)

        def step(t, p, cprev):
            u = u0 + t
            live = u < u1
            c = u // NBLK
            blk = u % NBLK
            rg = blk // BT
            bt = blk % BT

            @pl.when(live & (c != cprev))
            def _():
                pltpu.sync_copy(
                    wt_hbm.at[pl.ds(c * (ECH * VPAD), ECH * VPAD)], wt
                )

            @pl.when(live)
            def _():
                idx_wait(p)

                @pl.when(u + 1 < u1)
                def _():
                    idx_start(u + 1, p ^ 1)

                @pl.when(t >= 2)
                def _():
                    out_wait(p)

                @plsc.parallel_loop(0, 64)
                def col(sj):
                    s = sj // 8
                    j = sj % 8
                    iv = ibs[p][s, pl.ds(j * L, L)]
                    vals = [
                        plsc.load_gather(wt.at[pl.ds(e * VPAD, VPAD)], [iv])
                        for e in range(ECH)
                    ]
                    for e in range(ECH):
                        bufs[p][e, s, pl.ds(j * L, L)] = vals[e]

                pltpu.async_copy(
                    bufs[p],
                    out_hbm.at[
                        pl.ds(c * ECH, ECH),
                        pl.ds(rg * 8, 8),
                        pl.ds(bt * 128, 128),
                    ],
                    osem[p],
                )

            return jnp.where(live, c, cprev)

        def pair(i, cprev):
            cprev = step(i * 2, 0, cprev)
            cprev = step(i * 2 + 1, 1, cprev)
            return cprev

        lax.fori_loop(0, (cnt + 1) // 2, pair, jnp.int32(-1))
        out_wait(0)
        out_wait(1)

    return k(xt, wt_flat)


def kernel(x, W):
    xt = x.T
    wt = jnp.pad(W.T, ((0, 0), (0, VPAD - VOCAB)))
    wt_flat = wt.reshape(-1)
    r = _sc_kernel(xt, wt_flat)
    return jnp.transpose(r, (2, 1, 0))


# speedup vs baseline: 1.2907x; 1.0000x over previous
import functools

import jax
import jax.numpy as jnp
from jax import lax
from jax.experimental import pallas as pl
from jax.experimental.pallas import tpu as pltpu
from jax.experimental.pallas import tpu_sc as plsc

NC, NS = 2, 16
NW = NC * NS
BATCH, SEQ, EMBED = 1024, 200, 300
VOCAB = 1000
VPAD = 1024
ECH = 30
NECH = EMBED // ECH
BT = BATCH // 128
NBLK = (SEQ // 8) * BT
NUNIT = NBLK * NECH
L = 16


def _sc_kernel(xt, wt_flat):
    mesh = plsc.VectorSubcoreMesh(core_axis_name="c", subcore_axis_name="s")

    @functools.partial(
        pl.kernel,
        out_type=jax.ShapeDtypeStruct((EMBED, SEQ, BATCH), jnp.float32),
        mesh=mesh,
        scratch_types=[
            pltpu.VMEM((8, 128), jnp.int32),
            pltpu.VMEM((8, 128), jnp.int32),
            pltpu.VMEM((ECH * VPAD,), jnp.float32),
            pltpu.VMEM((ECH, 8, 128), jnp.float32),
            pltpu.VMEM((ECH, 8, 128), jnp.float32),
            pltpu.SemaphoreType.DMA,
            pltpu.SemaphoreType.DMA,
            pltpu.SemaphoreType.DMA,
            pltpu.SemaphoreType.DMA,
        ],
        compiler_params=pltpu.CompilerParams(
            use_tc_tiling_on_sc=True, needs_layout_passes=False,
            disable_bounds_checks=True
        ),
    )
    def k(xt_hbm, wt_hbm, out_hbm, ib0, ib1, wt, buf0, buf1,
          isem0, isem1, osem0, osem1):
        wid = lax.axis_index("s") * NC + lax.axis_index("c")
        u0 = wid * NUNIT // NW
        u1 = (wid + 1) * NUNIT // NW
        cnt = u1 - u0
        ibs = (ib0, ib1)
        isem = (isem0, isem1)
        bufs = (buf0, buf1)
        osem = (osem0, osem1)

        def idx_start(u, p):
            blk = u % NBLK
            rg = blk // BT
            bt = blk % BT
            pltpu.async_copy(
                xt_hbm.at[pl.ds(rg * 8, 8), pl.ds(bt * 128, 128)],
                ibs[p], isem[p],
            )

        def idx_wait(p):
            pltpu.make_async_copy(
                xt_hbm.at[pl.ds(0, 8), pl.ds(0, 128)], ibs[p], isem[p]
            ).wait()

        def out_wait(p):
            pltpu.make_async_copy(
                bufs[p],
                out_hbm.at[pl.ds(0, ECH), pl.ds(0, 8), pl.ds(0, 128)],
                osem[p],
            ).wait()

        idx_start(u0, 0)

        def step(t, p, cprev):
            u = u0 + t
            live = u < u1
            c = u // NBLK
            blk = u % NBLK
            rg = blk // BT
            bt = blk % BT

            @pl.when(live & (c != cprev))
            def _():
                pltpu.sync_copy(
                    wt_hbm.at[pl.ds(c * (ECH * VPAD), ECH * VPAD)], wt
                )

            @pl.when(live)
            def _():
                idx_wait(p)

                @pl.when(u + 1 < u1)
                def _():
                    idx_start(u + 1, p ^ 1)

                @pl.when(t >= 2)
                def _():
                    out_wait(p)

                @plsc.parallel_loop(0, 64)
                def col(sj):
                    s = sj // 8
                    j = sj % 8
                    iv = ibs[p][s, pl.ds(j * L, L)]
                    vals = [
                        plsc.load_gather(wt.at[pl.ds(e * VPAD, VPAD)], [iv])
                        for e in range(ECH)
                    ]
                    for e in range(ECH):
                        bufs[p][e, s, pl.ds(j * L, L)] = vals[e]

                pltpu.async_copy(
                    bufs[p],
                    out_hbm.at[
                        pl.ds(c * ECH, ECH),
                        pl.ds(rg * 8, 8),
                        pl.ds(bt * 128, 128),
                    ],
                    osem[p],
                )

            return jnp.where(live, c, cprev)

        def pair(i, cprev):
            cprev = step(i * 2, 0, cprev)
            cprev = step(i * 2 + 1, 1, cprev)
            return cprev

        lax.fori_loop(0, (cnt + 1) // 2, pair, jnp.int32(-1))
        out_wait(0)
        out_wait(1)

    return k(xt, wt_flat)


def kernel(x, W):
    xt = x.T
    wt = jnp.pad(W.T, ((0, 0), (0, VPAD - VOCAB)))
    wt_flat = wt.reshape(-1)
    r = _sc_kernel(xt, wt_flat)
    return jnp.transpose(r, (2, 1, 0))
